# baseline (speedup 1.0000x reference)
import functools
import jax
import jax.numpy as jnp
from jax import lax
from jax.experimental import pallas as pl
from jax.experimental.pallas import tpu as pltpu, tpu_sc as plsc

_NC, _NS = 2, 16
_NW = _NC * _NS


def kernel(x, embed_weight):
    del embed_weight
    b, s, d = x.shape
    rows = b * s
    x2 = x.reshape(rows, d)
    rows_per_w = rows // _NW

    mesh = plsc.VectorSubcoreMesh(core_axis_name="c", subcore_axis_name="s")

    @functools.partial(
        pl.kernel,
        mesh=mesh,
        out_type=jax.ShapeDtypeStruct((rows, d), jnp.float32),
    )
    def _sc_copy(in_hbm, out_hbm):
        wid = lax.axis_index("s") * _NC + lax.axis_index("c")
        base = wid * rows_per_w
        pltpu.sync_copy(
            in_hbm.at[pl.ds(base, rows_per_w), :],
            out_hbm.at[pl.ds(base, rows_per_w), :],
        )

    out = _sc_copy(x2)
    return out.reshape(b, s, d)


# speedup vs baseline: 23.3159x; 23.3159x over previous
import functools
import jax
import jax.numpy as jnp
from jax import lax
from jax.experimental import pallas as pl
from jax.experimental.pallas import tpu as pltpu, tpu_sc as plsc

_NC, _NS = 2, 16
_NW = _NC * _NS
_CHUNK = 32
_NBUF = 2


def kernel(x, embed_weight):
    del embed_weight
    b, s, d = x.shape
    rows = b * s
    x2 = x.reshape(rows, d)
    rows_per_w = rows // _NW
    nchunks = rows_per_w // _CHUNK

    mesh = plsc.VectorSubcoreMesh(core_axis_name="c", subcore_axis_name="s")

    @functools.partial(
        pl.kernel,
        mesh=mesh,
        out_type=jax.ShapeDtypeStruct((rows, d), jnp.float32),
        scratch_types=[
            pltpu.VMEM((_NBUF, _CHUNK, d), jnp.float32),
            pltpu.SemaphoreType.DMA((_NBUF,)),
        ],
    )
    def _sc_copy(in_hbm, out_hbm, buf, sems):
        wid = lax.axis_index("s") * _NC + lax.axis_index("c")
        base = wid * rows_per_w

        in_copies = [
            pltpu.make_async_copy(
                in_hbm.at[pl.ds(base + c * _CHUNK, _CHUNK), :],
                buf.at[c % _NBUF],
                sems.at[c % _NBUF],
            )
            for c in range(nchunks)
        ]
        in_copies[0].start()
        for c in range(nchunks):
            in_copies[c].wait()
            if c + 1 < nchunks:
                in_copies[c + 1].start()
            pltpu.sync_copy(
                buf.at[c % _NBUF],
                out_hbm.at[pl.ds(base + c * _CHUNK, _CHUNK), :],
            )

    out = _sc_copy(x2)
    return out.reshape(b, s, d)


# speedup vs baseline: 47.1260x; 2.0212x over previous
import jax
import jax.numpy as jnp
from jax.experimental import pallas as pl
from jax.experimental.pallas import tpu as pltpu

_GRID = 2
_CHUNK = 512
_NBUF = 8


def _relay_body(in_hbm, out_hbm, buf, in_sems, out_sems):
    rows = in_hbm.shape[0]
    per_core = rows // _GRID
    nchunks = per_core // _CHUNK
    base = pl.program_id(0) * per_core

    in_copies = [
        pltpu.make_async_copy(
            in_hbm.at[pl.ds(base + c * _CHUNK, _CHUNK), :],
            buf.at[c % _NBUF],
            in_sems.at[c % _NBUF],
        )
        for c in range(nchunks)
    ]
    out_copies = [
        pltpu.make_async_copy(
            buf.at[c % _NBUF],
            out_hbm.at[pl.ds(base + c * _CHUNK, _CHUNK), :],
            out_sems.at[c % _NBUF],
        )
        for c in range(nchunks)
    ]

    for c in range(min(_NBUF, nchunks)):
        in_copies[c].start()
    for c in range(nchunks):
        in_copies[c].wait()
        out_copies[c].start()
        if c + _NBUF < nchunks:
            out_copies[c].wait()
            in_copies[c + _NBUF].start()
    for c in range(max(0, nchunks - _NBUF), nchunks):
        out_copies[c].wait()


def kernel(x, embed_weight):
    del embed_weight
    b, s, d = x.shape
    rows = b * s
    x2 = x.reshape(rows, d)
    out = pl.pallas_call(
        _relay_body,
        out_shape=jax.ShapeDtypeStruct((rows, d), x.dtype),
        grid=(_GRID,),
        in_specs=[pl.BlockSpec(memory_space=pltpu.MemorySpace.HBM)],
        out_specs=pl.BlockSpec(memory_space=pltpu.MemorySpace.HBM),
        scratch_shapes=[
            pltpu.VMEM((_NBUF, _CHUNK, d), jnp.float32),
            pltpu.SemaphoreType.DMA((_NBUF,)),
            pltpu.SemaphoreType.DMA((_NBUF,)),
        ],
        compiler_params=pltpu.CompilerParams(
            dimension_semantics=("parallel",),
        ),
    )(x2)
    return out.reshape(b, s, d)


# speedup vs baseline: 49.9990x; 1.0610x over previous
import jax
import jax.numpy as jnp
from jax.experimental import pallas as pl
from jax.experimental.pallas import tpu as pltpu

_BLOCK_ROWS = 2048


def _copy_body(in_ref, out_ref):
    out_ref[...] = in_ref[...]


def kernel(x, embed_weight):
    del embed_weight
    b, s, d = x.shape
    rows = b * s
    x2 = x.reshape(rows, d)
    block_rows = min(_BLOCK_ROWS, rows)
    out = pl.pallas_call(
        _copy_body,
        out_shape=jax.ShapeDtypeStruct((rows, d), x.dtype),
        grid=(pl.cdiv(rows, block_rows),),
        in_specs=[pl.BlockSpec((block_rows, d), lambda i: (i, 0))],
        out_specs=pl.BlockSpec((block_rows, d), lambda i: (i, 0)),
        compiler_params=pltpu.CompilerParams(
            dimension_semantics=("parallel",),
        ),
    )(x2)
    return out.reshape(b, s, d)
